# Initial kernel scaffold; baseline (speedup 1.0000x reference)
import jax
import jax.numpy as jnp
from jax.experimental import pallas as pl


def kernel(action_idx, is_ground, physics_params, action_emb, W1, b1, W2, b2, W3, b3, gravity):
    raise NotImplementedError("write your pallas kernel here")


# speedup vs baseline: 2.2033x; 2.2033x over previous
import functools

import jax
import jax.numpy as jnp
from jax import lax
from jax.experimental import pallas as pl
from jax.experimental.pallas import tpu as pltpu
from jax.experimental.pallas import tpu_sc as plsc

_TBL_W = 16


def _sc_gather(table, idx):
    V, D = table.shape
    B = idx.shape[0]
    info = plsc.get_sparse_core_info()
    nw = info.num_cores * info.num_subcores
    b_per_w = B // nw
    mesh = plsc.VectorSubcoreMesh(core_axis_name="c", subcore_axis_name="s")

    @functools.partial(
        pl.kernel,
        mesh=mesh,
        compiler_params=pltpu.CompilerParams(use_tc_tiling_on_sc=False),
        out_type=jax.ShapeDtypeStruct((B, D), jnp.float32),
        scratch_types=[
            pltpu.VMEM((b_per_w,), jnp.int32),
            pltpu.VMEM((b_per_w, D), jnp.float32),
            pltpu.SemaphoreType.DMA,
        ],
    )
    def gather_kernel(table_hbm, idx_hbm, out_hbm, idx_v, rows_v, sem):
        wid = lax.axis_index("s") * info.num_cores + lax.axis_index("c")
        base = wid * b_per_w
        pltpu.sync_copy(idx_hbm.at[pl.ds(base, b_per_w)], idx_v)
        pltpu.async_copy(table_hbm.at[idx_v], rows_v, sem).wait()
        pltpu.sync_copy(rows_v, out_hbm.at[pl.ds(base, b_per_w)])

    return gather_kernel(table, idx)


def _tc_mlp(g, ig, w1e, wig, b1, w2t, b2, w3t, b3, sel):
    B = g.shape[0]
    blk = 2048
    grid = (B // blk,)

    def body(g_ref, ig_ref, w1e_ref, wig_ref, b1_ref, w2t_ref, b2_ref,
             w3t_ref, b3_ref, sel_ref, out_ref):
        x = g_ref[...]
        h = jnp.dot(x, w1e_ref[...], preferred_element_type=jnp.float32)
        h = jnp.maximum(h + ig_ref[...] * wig_ref[...] + b1_ref[...], 0.0)
        h = jnp.dot(h, w2t_ref[...], preferred_element_type=jnp.float32)
        h = jnp.maximum(h + b2_ref[...], 0.0)
        res = jnp.dot(h, w3t_ref[...], preferred_element_type=jnp.float32)
        base = jnp.dot(x, sel_ref[...], preferred_element_type=jnp.float32)
        out_ref[...] = base + res + b3_ref[...]

    full = lambda shape: pl.BlockSpec(shape, lambda i: (0, 0))
    return pl.pallas_call(
        body,
        grid=grid,
        in_specs=[
            pl.BlockSpec((blk, _TBL_W), lambda i: (i, 0)),
            pl.BlockSpec((blk, 1), lambda i: (i, 0)),
            full((_TBL_W, 32)),
            full((1, 32)),
            full((1, 32)),
            full((32, 16)),
            full((1, 16)),
            full((16, 2)),
            full((1, 2)),
            full((_TBL_W, 2)),
        ],
        out_specs=pl.BlockSpec((blk, 2), lambda i: (i, 0)),
        out_shape=jax.ShapeDtypeStruct((B, 2), jnp.float32),
    )(g, ig, w1e, wig, b1, w2t, b2, w3t, b3, sel)


def kernel(action_idx, is_ground, physics_params, action_emb,
           W1, b1, W2, b2, W3, b3, gravity):
    B = action_idx.shape[0]
    V = physics_params.shape[0]
    idx = action_idx.astype(jnp.int32)

    table = jnp.concatenate(
        [physics_params[:, :2], action_emb,
         jnp.zeros((V, _TBL_W - 10), jnp.float32)], axis=1)

    g = _sc_gather(table, idx)

    w1e = jnp.zeros((_TBL_W, 32), jnp.float32).at[2:10, :].set(W1[:, :8].T)
    wig = W1[:, 8].reshape(1, 32)
    sel = jnp.zeros((_TBL_W, 2), jnp.float32).at[0, 0].set(1.0).at[1, 1].set(1.0)

    out = _tc_mlp(g, is_ground.reshape(B, 1), w1e, wig, b1.reshape(1, 32),
                  W2.T, b2.reshape(1, 16), W3.T, b3.reshape(1, 2), sel)
    return (out, gravity)
